# baseline (speedup 1.0000x reference)
import jax
import jax.numpy as jnp
from jax.experimental import pallas as pl
from jax.experimental.pallas import tpu as pltpu

H = 256
K = 204
T_BLK = 2048
B_BLK = 4


def _fused_body(x_ref, wc_ref, wr_ref, mx_ref, bm_ref):
    sig_col = jax.nn.sigmoid(wc_ref[...])
    sig_row = jax.nn.sigmoid(wr_ref[...])
    col = jnp.broadcast_to(sig_col, (H, H))
    row = jnp.broadcast_to(sig_row, (H, H))
    cnt_col = jnp.sum((row > col).astype(jnp.float32), axis=1, keepdims=True)
    mask_col = (cnt_col < float(K)).astype(jnp.float32)
    cnt_row = jnp.sum((col > row).astype(jnp.float32), axis=0, keepdims=True)
    mask_row = (cnt_row < float(K)).astype(jnp.float32)

    mx_ref[...] = x_ref[...] * mask_col[None, None]
    bm_ref[...] = jnp.broadcast_to(mask_row[None, None], (B_BLK, 1, T_BLK, H))


def kernel(x, W):
    b, c, nfreq, ntime = x.shape

    grid = (b // B_BLK, ntime // T_BLK)
    masked_x, binary_mask = pl.pallas_call(
        _fused_body,
        grid=grid,
        in_specs=[
            pl.BlockSpec((B_BLK, 1, nfreq, T_BLK), lambda i, j: (i, 0, 0, j)),
            pl.BlockSpec((H, 1), lambda i, j: (0, 0)),
            pl.BlockSpec((1, H), lambda i, j: (0, 0)),
        ],
        out_specs=(
            pl.BlockSpec((B_BLK, 1, nfreq, T_BLK), lambda i, j: (i, 0, 0, j)),
            pl.BlockSpec((B_BLK, 1, T_BLK, H), lambda i, j: (i, 0, j, 0)),
        ),
        out_shape=(
            jax.ShapeDtypeStruct((b, c, nfreq, ntime), x.dtype),
            jax.ShapeDtypeStruct((b, c, ntime, nfreq), x.dtype),
        ),
        compiler_params=pltpu.CompilerParams(
            dimension_semantics=("parallel", "parallel"),
        ),
    )(x, W.reshape(H, 1), W.reshape(1, H))

    return masked_x, binary_mask


# speedup vs baseline: 1.0011x; 1.0011x over previous
import jax
import jax.numpy as jnp
from jax.experimental import pallas as pl
from jax.experimental.pallas import tpu as pltpu

H = 256
K = 204
T_BLK = 512
B_BLK = 16


def _fused_body(x_ref, wc_ref, wr_ref, mx_ref, bm_ref):
    sig_col = jax.nn.sigmoid(wc_ref[...])
    sig_row = jax.nn.sigmoid(wr_ref[...])
    col = jnp.broadcast_to(sig_col, (H, H))
    row = jnp.broadcast_to(sig_row, (H, H))
    cnt_col = jnp.sum((row > col).astype(jnp.float32), axis=1, keepdims=True)
    mask_col = (cnt_col < float(K)).astype(jnp.float32)
    cnt_row = jnp.sum((col > row).astype(jnp.float32), axis=0, keepdims=True)
    mask_row = (cnt_row < float(K)).astype(jnp.float32)

    mx_ref[...] = x_ref[...] * mask_col[None, None]
    bm_ref[...] = jnp.broadcast_to(mask_row[None, None], (B_BLK, 1, T_BLK, H))


def kernel(x, W):
    b, c, nfreq, ntime = x.shape

    grid = (b // B_BLK, ntime // T_BLK)
    masked_x, binary_mask = pl.pallas_call(
        _fused_body,
        grid=grid,
        in_specs=[
            pl.BlockSpec((B_BLK, 1, nfreq, T_BLK), lambda i, j: (i, 0, 0, j)),
            pl.BlockSpec((H, 1), lambda i, j: (0, 0)),
            pl.BlockSpec((1, H), lambda i, j: (0, 0)),
        ],
        out_specs=(
            pl.BlockSpec((B_BLK, 1, nfreq, T_BLK), lambda i, j: (i, 0, 0, j)),
            pl.BlockSpec((B_BLK, 1, T_BLK, H), lambda i, j: (i, 0, j, 0)),
        ),
        out_shape=(
            jax.ShapeDtypeStruct((b, c, nfreq, ntime), x.dtype),
            jax.ShapeDtypeStruct((b, c, ntime, nfreq), x.dtype),
        ),
        compiler_params=pltpu.CompilerParams(
            dimension_semantics=("parallel", "parallel"),
        ),
    )(x, W.reshape(H, 1), W.reshape(1, H))

    return masked_x, binary_mask
